# baseline (speedup 1.0000x reference)
import jax
import jax.numpy as jnp
from jax.experimental import pallas as pl
from jax.experimental.pallas import tpu as pltpu

_N = 6422528
_NB = 8
_BN = _N // _NB


def _fill_body(o_hbm, z, sem):
    z[...] = jnp.zeros((_BN,), jnp.float32)
    copies = [
        pltpu.make_async_copy(z, o_hbm.at[pl.ds(i * _BN, _BN)], sem)
        for i in range(_NB)
    ]
    for c in copies:
        c.start()
    for c in copies:
        c.wait()


def kernel(x):
    del x
    return pl.pallas_call(
        _fill_body,
        out_specs=pl.BlockSpec(memory_space=pltpu.MemorySpace.HBM),
        out_shape=jax.ShapeDtypeStruct((_N,), jnp.float32),
        scratch_shapes=[
            pltpu.VMEM((_BN,), jnp.float32),
            pltpu.SemaphoreType.DMA,
        ],
    )()


# speedup vs baseline: 1.0344x; 1.0344x over previous
import jax
import jax.numpy as jnp
from jax.experimental import pallas as pl

_N = 6422528
_BN = _N // 8


def _fill_body(o_ref):
    o_ref[...] = jnp.zeros((_BN,), jnp.float32)


def kernel(x):
    del x
    return pl.pallas_call(
        _fill_body,
        grid=(8,),
        out_specs=pl.BlockSpec((_BN,), lambda i: (i,)),
        out_shape=jax.ShapeDtypeStruct((_N,), jnp.float32),
    )()
